# baseline (speedup 1.0000x reference)
import functools

import jax
import jax.numpy as jnp
import numpy as np
from jax.experimental import pallas as pl

N = 10000
IN_CH = 3
EMB_DIM = 8
HID = 8
KNN = 50
R = 100.0
N_ITERS = 4
CHUNK = 1000


def _ln(h, g, b):
    m = jnp.mean(h, axis=-1, keepdims=True)
    v = jnp.var(h, axis=-1, keepdims=True)
    return (h - m) / jnp.sqrt(v + 1e-5) * g + b


def _emb_apply(p, x):
    for (W, b) in p["layers"]:
        x = jnp.tanh(x @ W + b)
    W, b = p["emb"]
    return x @ W + b


def _edge_net(p, x, start, end):
    h = jnp.concatenate([x[start], x[end]], axis=1)
    for i in range(3):
        W, b = p["lin"][i]
        g, be = p["ln"][i]
        h = jnp.tanh(_ln(h @ W + b, g, be))
    W, b = p["lin"][3]
    return (h @ W + b)[:, 0]


def _node_net(p, x, e, start, end, mask):
    ew = e * mask
    mi = jax.ops.segment_sum(ew[:, None] * x[start], end, num_segments=x.shape[0])
    mo = jax.ops.segment_sum(ew[:, None] * x[end], start, num_segments=x.shape[0])
    h = jnp.concatenate([mi, mo, x], axis=1)
    for i in range(3):
        W, b = p["lin"][i]
        g, be = p["ln"][i]
        h = jnp.tanh(_ln(h @ W + b, g, be))
    W, b = p["lin"][3]
    return h @ W + b


def _build_edges(spatial, layers):
    n = spatial.shape[0]
    sq = jnp.sum(spatial * spatial, axis=1)
    idxs, ds = [], []
    for s in range(0, n, CHUNK):
        q = spatial[s:s + CHUNK]
        d = jnp.sum(q * q, axis=1)[:, None] - 2.0 * (q @ spatial.T) + sq[None, :]
        nd, ni = jax.lax.top_k(-d, KNN)
        idxs.append(ni)
        ds.append(-nd)
    idx = jnp.concatenate(idxs, axis=0)
    dist = jnp.concatenate(ds, axis=0)
    end = jnp.repeat(jnp.arange(n), KNN)
    start = idx.reshape(-1)
    mask = (dist.reshape(-1) < R * R) & ((layers[end] - layers[start]) == 1)
    return start, end, mask.astype(jnp.float32)


def _edge_mlp_kernel(h_ref, m_ref, w0, b0, g0, be0, w1, b1, g1, be1,
                     w2, b2, g2, be2, w3, b3, o_ref):
    h = h_ref[...]
    for (W, b, g, be) in ((w0, b0, g0, be0), (w1, b1, g1, be1), (w2, b2, g2, be2)):
        h = jnp.dot(h, W[...], preferred_element_type=jnp.float32) + b[...][None, :]
        mu = jnp.mean(h, axis=-1, keepdims=True)
        v = jnp.var(h, axis=-1, keepdims=True)
        h = jnp.tanh((h - mu) / jnp.sqrt(v + 1e-5) * g[...][None, :] + be[...][None, :])
    h = jnp.dot(h, w3[...], preferred_element_type=jnp.float32) + b3[...][None, :]
    o_ref[...] = (h[:, 0] * m_ref[0, 0, :])[None, None, :]


def _edge_logits_pallas(p, h0, mask):
    E = h0.shape[0]
    BLK = 5000
    G = E // BLK
    flat = []
    for i in range(3):
        W, b = p["lin"][i]
        g, be = p["ln"][i]
        flat += [W, b, g, be]
    W3, b3 = p["lin"][3]
    flat += [W3, b3]
    specs = [pl.BlockSpec((BLK, 16), lambda i: (i, 0)),
             pl.BlockSpec((1, 1, BLK), lambda i: (i, 0, 0))]
    for a in flat:
        if a.ndim == 2:
            specs.append(pl.BlockSpec(a.shape, lambda i: (0, 0)))
        else:
            specs.append(pl.BlockSpec(a.shape, lambda i: (0,)))
    out = pl.pallas_call(
        _edge_mlp_kernel,
        grid=(G,),
        in_specs=specs,
        out_specs=pl.BlockSpec((1, 1, BLK), lambda i: (i, 0, 0)),
        out_shape=jax.ShapeDtypeStruct((G, 1, BLK), jnp.float32),
    )(h0, mask.reshape(G, 1, BLK), *flat)
    return out.reshape(E)


def kernel(x, layers, params):
    spatial = _emb_apply(params["emb1"], x)
    s1, e1, m1 = _build_edges(spatial, layers)
    W, b = params["ifn"]["lin"][0]
    g, be = params["ifn"]["ln"][0]
    f = jnp.tanh(_ln(jnp.concatenate([spatial, x], axis=-1) @ W + b, g, be))
    for _ in range(N_ITERS // 2):
        f0 = f
        e = jax.nn.sigmoid(_edge_net(params["edge"], f, s1, e1))
        f = _node_net(params["node"], f, e, s1, e1, m1) + f0
    spatial2 = _emb_apply(params["emb2"], jnp.concatenate([spatial, x, f], axis=-1))
    s2, e2, m2 = _build_edges(spatial2, layers)
    for _ in range(N_ITERS // 2):
        f0 = f
        e = jax.nn.sigmoid(_edge_net(params["edge"], f, s2, e2))
        f = _node_net(params["node"], f, e, s2, e2, m2) + f0
    h0 = jnp.concatenate([f[s2], f[e2]], axis=1)
    logits = _edge_logits_pallas(params["edge"], h0, m2)
    ratio = jnp.sum(m2) / spatial2.shape[0]
    return logits, spatial2, jnp.stack([s2, e2]), ratio


# speedup vs baseline: 1.9924x; 1.9924x over previous
import functools

import jax
import jax.numpy as jnp
import numpy as np
from jax.experimental import pallas as pl
from jax.experimental.pallas import tpu as pltpu

N = 10000
IN_CH = 3
EMB_DIM = 8
HID = 8
KNN = 50
R = 100.0
N_ITERS = 4
CHUNK = 1000


def _ln(h, g, b):
    m = jnp.mean(h, axis=-1, keepdims=True)
    v = jnp.var(h, axis=-1, keepdims=True)
    return (h - m) / jnp.sqrt(v + 1e-5) * g + b


def _emb_apply(p, x):
    for (W, b) in p["layers"]:
        x = jnp.tanh(x @ W + b)
    W, b = p["emb"]
    return x @ W + b


def _edge_net(p, x, start, end):
    h = jnp.concatenate([x[start], x[end]], axis=1)
    for i in range(3):
        W, b = p["lin"][i]
        g, be = p["ln"][i]
        h = jnp.tanh(_ln(h @ W + b, g, be))
    W, b = p["lin"][3]
    return (h @ W + b)[:, 0]


def _node_net(p, x, e, start, end, mask):
    ew = e * mask
    mi = jax.ops.segment_sum(ew[:, None] * x[start], end, num_segments=x.shape[0])
    mo = jax.ops.segment_sum(ew[:, None] * x[end], start, num_segments=x.shape[0])
    h = jnp.concatenate([mi, mo, x], axis=1)
    for i in range(3):
        W, b = p["lin"][i]
        g, be = p["ln"][i]
        h = jnp.tanh(_ln(h @ W + b, g, be))
    W, b = p["lin"][3]
    return h @ W + b


_KNN_B = 400
_KNN_NPAD = 10240


def _knn_kernel(qsq_ref, q_ref, st_ref, sqp_ref, dist_ref, idx_ref, s_ref):
    B = q_ref.shape[0]
    npad = sqp_ref.shape[1]
    mm = jnp.dot(q_ref[...], st_ref[...], preferred_element_type=jnp.float32)
    d = qsq_ref[...] - 2.0 * mm + sqp_ref[...]
    s_ref[...] = d
    iota = jax.lax.broadcasted_iota(jnp.int32, (B, npad), 1)
    kio = jax.lax.broadcasted_iota(jnp.int32, (B, KNN), 1)
    big = jnp.int32(2**30)

    def body(it, carry):
        acc_d, acc_i = carry
        S = s_ref[...]
        v = jnp.min(S, axis=1, keepdims=True)
        cand = jnp.where(S == v, iota, big)
        i = jnp.min(cand, axis=1, keepdims=True)
        s_ref[...] = jnp.where(iota == i, jnp.inf, S)
        sel = (kio == it)
        acc_d = jnp.where(sel, v, acc_d)
        acc_i = jnp.where(sel, i, acc_i)
        return acc_d, acc_i

    acc_d = jnp.zeros((B, KNN), jnp.float32)
    acc_i = jnp.zeros((B, KNN), jnp.int32)
    acc_d, acc_i = jax.lax.fori_loop(0, KNN, body, (acc_d, acc_i))
    dist_ref[...] = acc_d
    idx_ref[...] = acc_i


def _knn_pallas(spatial):
    n = spatial.shape[0]
    sq = jnp.sum(spatial * spatial, axis=1)
    qsq = sq[:, None]
    sqp = jnp.full((1, _KNN_NPAD), 1e30, jnp.float32).at[0, :n].set(sq)
    st = jnp.zeros((8, _KNN_NPAD), jnp.float32).at[:, :n].set(spatial.T)
    dist, idx = pl.pallas_call(
        _knn_kernel,
        grid=(n // _KNN_B,),
        in_specs=[
            pl.BlockSpec((_KNN_B, 1), lambda i: (i, 0)),
            pl.BlockSpec((_KNN_B, 8), lambda i: (i, 0)),
            pl.BlockSpec((8, _KNN_NPAD), lambda i: (0, 0)),
            pl.BlockSpec((1, _KNN_NPAD), lambda i: (0, 0)),
        ],
        out_specs=[
            pl.BlockSpec((_KNN_B, KNN), lambda i: (i, 0)),
            pl.BlockSpec((_KNN_B, KNN), lambda i: (i, 0)),
        ],
        out_shape=[
            jax.ShapeDtypeStruct((n, KNN), jnp.float32),
            jax.ShapeDtypeStruct((n, KNN), jnp.int32),
        ],
        scratch_shapes=[pltpu.VMEM((_KNN_B, _KNN_NPAD), jnp.float32)],
    )(qsq, spatial, st, sqp)
    return dist, idx


def _build_edges(spatial, layers):
    n = spatial.shape[0]
    dist, idx = _knn_pallas(spatial)
    end = jnp.repeat(jnp.arange(n), KNN)
    start = idx.reshape(-1)
    mask = (dist.reshape(-1) < R * R) & ((layers[end] - layers[start]) == 1)
    return start, end, mask.astype(jnp.float32)


def _edge_mlp_kernel(h_ref, m_ref, w0, b0, g0, be0, w1, b1, g1, be1,
                     w2, b2, g2, be2, w3, b3, o_ref):
    h = h_ref[...]
    for (W, b, g, be) in ((w0, b0, g0, be0), (w1, b1, g1, be1), (w2, b2, g2, be2)):
        h = jnp.dot(h, W[...], preferred_element_type=jnp.float32) + b[...][None, :]
        mu = jnp.mean(h, axis=-1, keepdims=True)
        v = jnp.var(h, axis=-1, keepdims=True)
        h = jnp.tanh((h - mu) / jnp.sqrt(v + 1e-5) * g[...][None, :] + be[...][None, :])
    h = jnp.dot(h, w3[...], preferred_element_type=jnp.float32) + b3[...][None, :]
    o_ref[...] = (h[:, 0] * m_ref[0, 0, :])[None, None, :]


def _edge_logits_pallas(p, h0, mask):
    E = h0.shape[0]
    BLK = 5000
    G = E // BLK
    flat = []
    for i in range(3):
        W, b = p["lin"][i]
        g, be = p["ln"][i]
        flat += [W, b, g, be]
    W3, b3 = p["lin"][3]
    flat += [W3, b3]
    specs = [pl.BlockSpec((BLK, 16), lambda i: (i, 0)),
             pl.BlockSpec((1, 1, BLK), lambda i: (i, 0, 0))]
    for a in flat:
        if a.ndim == 2:
            specs.append(pl.BlockSpec(a.shape, lambda i: (0, 0)))
        else:
            specs.append(pl.BlockSpec(a.shape, lambda i: (0,)))
    out = pl.pallas_call(
        _edge_mlp_kernel,
        grid=(G,),
        in_specs=specs,
        out_specs=pl.BlockSpec((1, 1, BLK), lambda i: (i, 0, 0)),
        out_shape=jax.ShapeDtypeStruct((G, 1, BLK), jnp.float32),
    )(h0, mask.reshape(G, 1, BLK), *flat)
    return out.reshape(E)


def kernel(x, layers, params):
    spatial = _emb_apply(params["emb1"], x)
    s1, e1, m1 = _build_edges(spatial, layers)
    W, b = params["ifn"]["lin"][0]
    g, be = params["ifn"]["ln"][0]
    f = jnp.tanh(_ln(jnp.concatenate([spatial, x], axis=-1) @ W + b, g, be))
    for _ in range(N_ITERS // 2):
        f0 = f
        e = jax.nn.sigmoid(_edge_net(params["edge"], f, s1, e1))
        f = _node_net(params["node"], f, e, s1, e1, m1) + f0
    spatial2 = _emb_apply(params["emb2"], jnp.concatenate([spatial, x, f], axis=-1))
    s2, e2, m2 = _build_edges(spatial2, layers)
    for _ in range(N_ITERS // 2):
        f0 = f
        e = jax.nn.sigmoid(_edge_net(params["edge"], f, s2, e2))
        f = _node_net(params["node"], f, e, s2, e2, m2) + f0
    h0 = jnp.concatenate([f[s2], f[e2]], axis=1)
    logits = _edge_logits_pallas(params["edge"], h0, m2)
    ratio = jnp.sum(m2) / spatial2.shape[0]
    return logits, spatial2, jnp.stack([s2, e2]), ratio


# speedup vs baseline: 2.2704x; 1.1396x over previous
import functools

import jax
import jax.numpy as jnp
import numpy as np
from jax import lax
from jax.experimental import pallas as pl
from jax.experimental.pallas import tpu as pltpu
from jax.experimental.pallas import tpu_sc as plsc

N = 10000
IN_CH = 3
EMB_DIM = 8
HID = 8
KNN = 50
R = 100.0
N_ITERS = 4
CHUNK = 1000


def _ln(h, g, b):
    m = jnp.mean(h, axis=-1, keepdims=True)
    v = jnp.var(h, axis=-1, keepdims=True)
    return (h - m) / jnp.sqrt(v + 1e-5) * g + b


def _emb_apply(p, x):
    for (W, b) in p["layers"]:
        x = jnp.tanh(x @ W + b)
    W, b = p["emb"]
    return x @ W + b


def _edge_net(p, x, start, end):
    h = jnp.concatenate([x[start], x[end]], axis=1)
    for i in range(3):
        W, b = p["lin"][i]
        g, be = p["ln"][i]
        h = jnp.tanh(_ln(h @ W + b, g, be))
    W, b = p["lin"][3]
    return (h @ W + b)[:, 0]


def _node_net(p, x, e, start, end, mask):
    ew = e * mask
    mi = jax.ops.segment_sum(ew[:, None] * x[start], end, num_segments=x.shape[0])
    mo = jax.ops.segment_sum(ew[:, None] * x[end], start, num_segments=x.shape[0])
    h = jnp.concatenate([mi, mo, x], axis=1)
    for i in range(3):
        W, b = p["lin"][i]
        g, be = p["ln"][i]
        h = jnp.tanh(_ln(h @ W + b, g, be))
    W, b = p["lin"][3]
    return h @ W + b


_KNN_B = 400
_KNN_NPAD = 10240


def _knn_kernel(qsq_ref, q_ref, st_ref, sqp_ref, dist_ref, idx_ref, s_ref):
    B = q_ref.shape[0]
    npad = sqp_ref.shape[1]
    mm = jnp.dot(q_ref[...], st_ref[...], preferred_element_type=jnp.float32)
    d = qsq_ref[...] - 2.0 * mm + sqp_ref[...]
    s_ref[...] = d
    iota = jax.lax.broadcasted_iota(jnp.int32, (B, npad), 1)
    kio = jax.lax.broadcasted_iota(jnp.int32, (B, KNN), 1)
    big = jnp.int32(2**30)

    def body(it, carry):
        acc_d, acc_i = carry
        S = s_ref[...]
        v = jnp.min(S, axis=1, keepdims=True)
        cand = jnp.where(S == v, iota, big)
        i = jnp.min(cand, axis=1, keepdims=True)
        s_ref[...] = jnp.where(iota == i, jnp.inf, S)
        sel = (kio == it)
        acc_d = jnp.where(sel, v, acc_d)
        acc_i = jnp.where(sel, i, acc_i)
        return acc_d, acc_i

    acc_d = jnp.zeros((B, KNN), jnp.float32)
    acc_i = jnp.zeros((B, KNN), jnp.int32)
    acc_d, acc_i = jax.lax.fori_loop(0, KNN, body, (acc_d, acc_i))
    dist_ref[...] = acc_d
    idx_ref[...] = acc_i


def _knn_pallas(spatial):
    n = spatial.shape[0]
    sq = jnp.sum(spatial * spatial, axis=1)
    qsq = sq[:, None]
    sqp = jnp.full((1, _KNN_NPAD), 1e30, jnp.float32).at[0, :n].set(sq)
    st = jnp.zeros((8, _KNN_NPAD), jnp.float32).at[:, :n].set(spatial.T)
    dist, idx = pl.pallas_call(
        _knn_kernel,
        grid=(n // _KNN_B,),
        in_specs=[
            pl.BlockSpec((_KNN_B, 1), lambda i: (i, 0)),
            pl.BlockSpec((_KNN_B, 8), lambda i: (i, 0)),
            pl.BlockSpec((8, _KNN_NPAD), lambda i: (0, 0)),
            pl.BlockSpec((1, _KNN_NPAD), lambda i: (0, 0)),
        ],
        out_specs=[
            pl.BlockSpec((_KNN_B, KNN), lambda i: (i, 0)),
            pl.BlockSpec((_KNN_B, KNN), lambda i: (i, 0)),
        ],
        out_shape=[
            jax.ShapeDtypeStruct((n, KNN), jnp.float32),
            jax.ShapeDtypeStruct((n, KNN), jnp.int32),
        ],
        scratch_shapes=[pltpu.VMEM((_KNN_B, _KNN_NPAD), jnp.float32)],
    )(qsq, spatial, st, sqp)
    return dist, idx


def _build_edges(spatial, layers):
    n = spatial.shape[0]
    dist, idx = _knn_pallas(spatial)
    end = jnp.repeat(jnp.arange(n), KNN)
    start = idx.reshape(-1)
    mask = (dist.reshape(-1) < R * R) & ((layers[end] - layers[start]) == 1)
    return start, end, mask.astype(jnp.float32)


_E_TOT = N * KNN
_SC_CH = 2000
_SC_NCH = _E_TOT // _SC_CH
_SC_NC = 2
_SC_NS = 16
_SC_NW = _SC_NC * _SC_NS


def _sc_gather_kernel(table_hbm, idx_hbm, out_hbm, idx_v, rows_v, ftab, sem):
    c = lax.axis_index("c")
    s = lax.axis_index("s")
    wid = s * _SC_NC + c

    @pl.when(s < 10)
    def _():
        pltpu.sync_copy(table_hbm.at[pl.ds(s * 1000, 1000)],
                        ftab.at[pl.ds(s * 1000, 1000)])

    plsc.subcore_barrier()
    for k in range(-(-_SC_NCH // _SC_NW)):
        cid = wid + _SC_NW * k

        @pl.when(cid < _SC_NCH)
        def _():
            base = cid * _SC_CH
            pltpu.sync_copy(idx_hbm.at[pl.ds(base, _SC_CH)], idx_v)
            pltpu.async_copy(ftab.at[idx_v], rows_v, sem).wait()
            pltpu.sync_copy(rows_v, out_hbm.at[pl.ds(base, _SC_CH)])


def _sc_gather(table, idx):
    mesh = plsc.VectorSubcoreMesh(core_axis_name="c", subcore_axis_name="s")
    return pl.kernel(
        _sc_gather_kernel,
        mesh=mesh,
        out_type=jax.ShapeDtypeStruct((_E_TOT, HID), jnp.float32),
        scratch_types=[
            pltpu.VMEM((_SC_CH,), jnp.int32),
            pltpu.VMEM((_SC_CH, HID), jnp.float32),
            pltpu.VMEM_SHARED((N, HID), jnp.float32),
            pltpu.SemaphoreType.DMA,
        ],
    )(table, idx)


def _sc_scatter_kernel(idx_hbm, g_hbm, zeros_hbm, out_hbm, idx_v, g_v, acc):
    c = lax.axis_index("c")
    s = lax.axis_index("s")
    wid = s * _SC_NC + c

    @pl.when(s < 10)
    def _():
        pltpu.sync_copy(zeros_hbm.at[pl.ds(s * 1000, 1000)],
                        acc.at[pl.ds(s * 1000, 1000)])

    plsc.subcore_barrier()
    for k in range(-(-_SC_NCH // _SC_NW)):
        cid = wid + _SC_NW * k

        @pl.when(cid < _SC_NCH)
        def _():
            base = cid * _SC_CH
            pltpu.sync_copy(idx_hbm.at[pl.ds(base, _SC_CH)], idx_v)
            pltpu.sync_copy(g_hbm.at[pl.ds(base, _SC_CH)], g_v)
            pltpu.sync_copy(g_v, acc.at[idx_v], add=True)

    plsc.subcore_barrier()

    @pl.when(s < 10)
    def _():
        pltpu.sync_copy(acc.at[pl.ds(s * 1000, 1000)],
                        out_hbm.at[c].at[pl.ds(s * 1000, 1000)])


def _sc_scatter_add(idx, g, zeros):
    mesh = plsc.VectorSubcoreMesh(core_axis_name="c", subcore_axis_name="s")
    return pl.kernel(
        _sc_scatter_kernel,
        mesh=mesh,
        out_type=jax.ShapeDtypeStruct((_SC_NC, N, HID), jnp.float32),
        scratch_types=[
            pltpu.VMEM((_SC_CH,), jnp.int32),
            pltpu.VMEM((_SC_CH, HID), jnp.float32),
            pltpu.VMEM_SHARED((N, HID), jnp.float32),
        ],
    )(idx, g, zeros)


_EB = 400


def _edge_stage_kernel(do_mi, fsT_ref, f_ref, m_ref, w1a, w1b, b1, g1, be1,
                       w2, b2, g2, be2, w3, b3, g3, be3, w4, b4, *outs):
    W1a = w1a[...]
    W2 = w2[...]
    W3 = w3[...]
    W4 = w4[...]
    fb = f_ref[...]
    z1 = jnp.dot(fb, w1b[...], preferred_element_type=jnp.float32) + b1[...][None, :]

    def layer_norm_tanh(hs, gv, bev):
        mu = hs[0]
        for t in hs[1:]:
            mu = mu + t
        mu = mu * (1.0 / HID)
        var = (hs[0] - mu) * (hs[0] - mu)
        for t in hs[1:]:
            var = var + (t - mu) * (t - mu)
        var = var * (1.0 / HID)
        rstd = jax.lax.rsqrt(var + 1e-5)
        return [jnp.tanh((hs[o] - mu) * rstd * gv[o] + bev[o]) for o in range(HID)]

    h1 = []
    for o in range(HID):
        t = z1[:, o:o + 1]
        for i in range(HID):
            t = t + fsT_ref[i] * W1a[i, o]
        h1.append(t)
    h1 = layer_norm_tanh(h1, g1[...], be1[...])
    h2 = []
    for o in range(HID):
        t = b2[...][o]
        for i in range(HID):
            t = t + h1[i] * W2[i, o]
        h2.append(t)
    h2 = layer_norm_tanh(h2, g2[...], be2[...])
    h3 = []
    for o in range(HID):
        t = b3[...][o]
        for i in range(HID):
            t = t + h2[i] * W3[i, o]
        h3.append(t)
    h3 = layer_norm_tanh(h3, g3[...], be3[...])
    e_arr = b4[...][0]
    for i in range(HID):
        e_arr = e_arr + h3[i] * W4[i, 0]
    if do_mi:
        ew_ref, mi_ref = outs
        ew = jax.nn.sigmoid(e_arr) * m_ref[...]
        ew_ref[...] = ew
        for c in range(HID):
            mi_ref[:, c:c + 1] = jnp.sum(ew * fsT_ref[c], axis=1, keepdims=True)
    else:
        (out_ref,) = outs
        out_ref[...] = e_arr * m_ref[...]


def _edge_stage(p, fsT3, f, m2r, do_mi):
    flat = []
    W1, b1 = p["lin"][0]
    flat += [W1[:HID], W1[HID:], b1, p["ln"][0][0], p["ln"][0][1]]
    for i in (1, 2):
        W, b = p["lin"][i]
        flat += [W, b, p["ln"][i][0], p["ln"][i][1]]
    W4, b4 = p["lin"][3]
    flat += [W4, b4]
    specs = [
        pl.BlockSpec((HID, _EB, KNN), lambda i: (0, i, 0)),
        pl.BlockSpec((_EB, HID), lambda i: (i, 0)),
        pl.BlockSpec((_EB, KNN), lambda i: (i, 0)),
    ]
    for a in flat:
        if a.ndim == 2:
            specs.append(pl.BlockSpec(a.shape, lambda i: (0, 0)))
        else:
            specs.append(pl.BlockSpec(a.shape, lambda i: (0,)))
    if do_mi:
        out_specs = [pl.BlockSpec((_EB, KNN), lambda i: (i, 0)),
                     pl.BlockSpec((_EB, HID), lambda i: (i, 0))]
        out_shape = [jax.ShapeDtypeStruct((N, KNN), jnp.float32),
                     jax.ShapeDtypeStruct((N, HID), jnp.float32)]
    else:
        out_specs = pl.BlockSpec((_EB, KNN), lambda i: (i, 0))
        out_shape = jax.ShapeDtypeStruct((N, KNN), jnp.float32)
    return pl.pallas_call(
        functools.partial(_edge_stage_kernel, do_mi),
        grid=(N // _EB,),
        in_specs=specs,
        out_specs=out_specs,
        out_shape=out_shape,
    )(fsT3, f, m2r, *flat)


def _node_stage_kernel(mi_ref, mo0_ref, mo1_ref, f_ref, wa, wb, wc, b1, g1, be1,
                       w2, b2, g2, be2, w3, b3, g3, be3, w4, b4, out_ref):
    f = f_ref[...]
    mo = mo0_ref[...] + mo1_ref[...]
    h = (jnp.dot(mi_ref[...], wa[...], preferred_element_type=jnp.float32)
         + jnp.dot(mo, wb[...], preferred_element_type=jnp.float32)
         + jnp.dot(f, wc[...], preferred_element_type=jnp.float32)
         + b1[...][None, :])
    for (g, be, w, b) in ((g1, be1, w2, b2), (g2, be2, w3, b3), (g3, be3, w4, b4)):
        mu = jnp.mean(h, axis=-1, keepdims=True)
        v = jnp.var(h, axis=-1, keepdims=True)
        h = jnp.tanh((h - mu) / jnp.sqrt(v + 1e-5) * g[...][None, :] + be[...][None, :])
        h = jnp.dot(h, w[...], preferred_element_type=jnp.float32) + b[...][None, :]
    out_ref[...] = h + f


def _node_stage(p, mi, mo0, mo1, f):
    W1, b1 = p["lin"][0]
    flat = [W1[:HID], W1[HID:2 * HID], W1[2 * HID:], b1]
    for i in (0, 1, 2):
        flat += [p["ln"][i][0], p["ln"][i][1]]
        W, b = p["lin"][i + 1]
        flat += [W, b]
    order = flat[:4] + [flat[4], flat[5], flat[6], flat[7],
                        flat[8], flat[9], flat[10], flat[11],
                        flat[12], flat[13], flat[14], flat[15]]
    specs = []
    for a in order:
        if a.ndim == 2:
            specs.append(pl.BlockSpec(a.shape, lambda i: (0, 0)))
        else:
            specs.append(pl.BlockSpec(a.shape, lambda i: (0,)))
    dat_specs = [pl.BlockSpec((N, HID), lambda i: (0, 0)) for _ in range(4)]
    return pl.pallas_call(
        _node_stage_kernel,
        grid=(1,),
        in_specs=dat_specs + specs,
        out_specs=pl.BlockSpec((N, HID), lambda i: (0, 0)),
        out_shape=jax.ShapeDtypeStruct((N, HID), jnp.float32),
    )(mi, mo0, mo1, f, *order)


def kernel(x, layers, params):
    spatial = _emb_apply(params["emb1"], x)
    s1, e1, m1 = _build_edges(spatial, layers)
    W, b = params["ifn"]["lin"][0]
    g, be = params["ifn"]["ln"][0]
    f = jnp.tanh(_ln(jnp.concatenate([spatial, x], axis=-1) @ W + b, g, be))
    for _ in range(N_ITERS // 2):
        f0 = f
        e = jax.nn.sigmoid(_edge_net(params["edge"], f, s1, e1))
        f = _node_net(params["node"], f, e, s1, e1, m1) + f0
    spatial2 = _emb_apply(params["emb2"], jnp.concatenate([spatial, x, f], axis=-1))
    s2, e2, m2 = _build_edges(spatial2, layers)
    m2r = m2.reshape(N, KNN)
    zeros_nh = jnp.zeros((N, HID), jnp.float32)
    for _ in range(N_ITERS // 2):
        f0 = f
        fs = f[s2]
        fsT3 = fs.T.reshape(HID, N, KNN)
        ew, mi = _edge_stage(params["edge"], fsT3, f, m2r, do_mi=True)
        gvals = ew.reshape(_E_TOT, 1) * jnp.repeat(f, KNN, axis=0)
        mop = jax.ops.segment_sum(gvals, s2, num_segments=N)
        f = _node_stage(params["node"], mi, mop, zeros_nh, f0)
    fs = f[s2]
    fsT3 = fs.T.reshape(HID, N, KNN)
    logits = _edge_stage(params["edge"], fsT3, f, m2r, do_mi=False).reshape(_E_TOT)
    ratio = jnp.sum(m2) / spatial2.shape[0]
    return logits, spatial2, jnp.stack([s2, e2]), ratio
